# baseline (speedup 1.0000x reference)
import math
from functools import partial

import jax
import jax.numpy as jnp
from jax import lax
from jax.experimental import pallas as pl
from jax.experimental.pallas import tpu as pltpu

_INV_SQRT2 = 1.0 / math.sqrt(2.0)
_NEG_INF = -1e30
_HEADS = 12
_PATCH = 16
_EPS = 1e-5


def _ru(x, m):
    return ((x + m - 1) // m) * m


def _vmem_limit(bytes_needed):
    return int(min(64 * 2**20, max(32 * 2**20, 2 * bytes_needed)))


def _ln_rows(xv, g, b):
    mean = jnp.mean(xv, axis=-1, keepdims=True)
    xc = xv - mean
    var = jnp.mean(xc * xc, axis=-1, keepdims=True)
    return xc * lax.rsqrt(var + _EPS) * g + b


def _embed_kernel(p_ref, w_ref, b_ref, pos_ref, cls_ref, o_ref, *, n_tok):
    t = jnp.dot(p_ref[0], w_ref[...], preferred_element_type=jnp.float32)
    y = t + b_ref[...] + pos_ref[...]
    rows = lax.broadcasted_iota(jnp.int32, (y.shape[0], 1), 0)
    y = jnp.where(rows == 0, cls_ref[...], y)
    y = jnp.where(rows >= n_tok, 0.0, y)
    o_ref[0] = y


def _mha(xv, qkv, mask, *, heads, hd, scale):
    dim = heads * hd
    outs = []
    for h in range(heads):
        q = qkv[:, h * hd:(h + 1) * hd].astype(jnp.bfloat16)
        k = qkv[:, dim + h * hd:dim + (h + 1) * hd].astype(jnp.bfloat16)
        v = qkv[:, 2 * dim + h * hd:2 * dim + (h + 1) * hd].astype(jnp.bfloat16)
        s = lax.dot_general(q, k, (((1,), (1,)), ((), ())),
                            preferred_element_type=jnp.float32) * scale
        s = jnp.where(mask, _NEG_INF, s)
        s = s - jnp.max(s, axis=-1, keepdims=True)
        p = jnp.exp(s)
        p = p / jnp.sum(p, axis=-1, keepdims=True)
        outs.append(jnp.dot(p.astype(jnp.bfloat16), v,
                            preferred_element_type=jnp.float32))
    return jnp.concatenate(outs, axis=1)


def _layer_kernel(x_ref, g1_ref, b1g_ref, wq_ref, bq_ref, pw_ref, pb_ref,
                  g2_ref, b2g_ref, w1_ref, b1_ref, w2_ref, b2_ref, out_ref,
                  *, n_tok, heads, hd, scale):
    xv = x_ref[0]
    ln = _ln_rows(xv, g1_ref[...], b1g_ref[...]).astype(jnp.bfloat16)
    qkv = jnp.dot(ln, wq_ref[...], preferred_element_type=jnp.float32)
    qkv = qkv + bq_ref[...]
    mask = lax.broadcasted_iota(jnp.int32, (xv.shape[0], xv.shape[0]), 1) >= n_tok
    o = _mha(xv, qkv, mask, heads=heads, hd=hd, scale=scale)
    t = jnp.dot(o.astype(jnp.bfloat16), pw_ref[...],
                preferred_element_type=jnp.float32) + pb_ref[...]
    xmid = xv + t
    ln2 = _ln_rows(xmid, g2_ref[...], b2g_ref[...]).astype(jnp.bfloat16)
    hh = jnp.dot(ln2, w1_ref[...], preferred_element_type=jnp.float32) + b1_ref[...]
    gl = 0.5 * hh * (1.0 + lax.erf(hh * _INV_SQRT2))
    m = jnp.dot(gl.astype(jnp.bfloat16), w2_ref[...],
                preferred_element_type=jnp.float32) + b2_ref[...]
    out_ref[0] = xmid + m


def _final_kernel(x_ref, g_ref, b_ref, o_ref):
    o_ref[...] = _ln_rows(x_ref[...], g_ref[...], b_ref[...])


def _row2d(a):
    return a.reshape(1, a.shape[-1]).astype(jnp.float32)


def kernel(patch_embed_w, patch_embed_b, cls_token, pos_embed, norm_g, norm_b, block0_ln1_g, block0_ln1_b, block0_qkv_w, block0_qkv_b, block0_proj_w, block0_proj_b, block0_ln2_g, block0_ln2_b, block0_fc1_w, block0_fc1_b, block0_fc2_w, block0_fc2_b, block1_ln1_g, block1_ln1_b, block1_qkv_w, block1_qkv_b, block1_proj_w, block1_proj_b, block1_ln2_g, block1_ln2_b, block1_fc1_w, block1_fc1_b, block1_fc2_w, block1_fc2_b, block2_ln1_g, block2_ln1_b, block2_qkv_w, block2_qkv_b, block2_proj_w, block2_proj_b, block2_ln2_g, block2_ln2_b, block2_fc1_w, block2_fc1_b, block2_fc2_w, block2_fc2_b, block3_ln1_g, block3_ln1_b, block3_qkv_w, block3_qkv_b, block3_proj_w, block3_proj_b, block3_ln2_g, block3_ln2_b, block3_fc1_w, block3_fc1_b, block3_fc2_w, block3_fc2_b, block4_ln1_g, block4_ln1_b, block4_qkv_w, block4_qkv_b, block4_proj_w, block4_proj_b, block4_ln2_g, block4_ln2_b, block4_fc1_w, block4_fc1_b, block4_fc2_w, block4_fc2_b, block5_ln1_g, block5_ln1_b, block5_qkv_w, block5_qkv_b, block5_proj_w, block5_proj_b, block5_ln2_g, block5_ln2_b, block5_fc1_w, block5_fc1_b, block5_fc2_w, block5_fc2_b, x):
    blocks = [
        (block0_ln1_g, block0_ln1_b, block0_qkv_w, block0_qkv_b, block0_proj_w,
         block0_proj_b, block0_ln2_g, block0_ln2_b, block0_fc1_w, block0_fc1_b,
         block0_fc2_w, block0_fc2_b),
        (block1_ln1_g, block1_ln1_b, block1_qkv_w, block1_qkv_b, block1_proj_w,
         block1_proj_b, block1_ln2_g, block1_ln2_b, block1_fc1_w, block1_fc1_b,
         block1_fc2_w, block1_fc2_b),
        (block2_ln1_g, block2_ln1_b, block2_qkv_w, block2_qkv_b, block2_proj_w,
         block2_proj_b, block2_ln2_g, block2_ln2_b, block2_fc1_w, block2_fc1_b,
         block2_fc2_w, block2_fc2_b),
        (block3_ln1_g, block3_ln1_b, block3_qkv_w, block3_qkv_b, block3_proj_w,
         block3_proj_b, block3_ln2_g, block3_ln2_b, block3_fc1_w, block3_fc1_b,
         block3_fc2_w, block3_fc2_b),
        (block4_ln1_g, block4_ln1_b, block4_qkv_w, block4_qkv_b, block4_proj_w,
         block4_proj_b, block4_ln2_g, block4_ln2_b, block4_fc1_w, block4_fc1_b,
         block4_fc2_w, block4_fc2_b),
        (block5_ln1_g, block5_ln1_b, block5_qkv_w, block5_qkv_b, block5_proj_w,
         block5_proj_b, block5_ln2_g, block5_ln2_b, block5_fc1_w, block5_fc1_b,
         block5_fc2_w, block5_fc2_b),
    ]

    B, C, IMG, _ = x.shape
    p = _PATCH
    gh = IMG // p
    n_patch = gh * gh
    n_tok = n_patch + 1
    n_pad = _ru(n_tok, 8)
    D = patch_embed_w.shape[1]
    K = C * p * p
    H = _HEADS
    hd = D // H
    hidden = blocks[0][8].shape[1]
    scale = hd ** -0.5

    patches = x.reshape(B, C, gh, p, gh, p).transpose(0, 2, 4, 1, 3, 5)
    patches = patches.reshape(B, n_patch, K)
    P = jnp.pad(patches, ((0, 0), (1, n_pad - n_tok), (0, 0))).astype(jnp.bfloat16)
    pos = pos_embed[0].astype(jnp.float32)
    pos_pad = jnp.pad(pos, ((0, n_pad - n_tok), (0, 0)))
    cls0 = (cls_token[0, 0] + pos[0]).reshape(1, D).astype(jnp.float32)

    xs = pl.pallas_call(
        partial(_embed_kernel, n_tok=n_tok),
        out_shape=jax.ShapeDtypeStruct((B, n_pad, D), jnp.float32),
        grid_spec=pltpu.PrefetchScalarGridSpec(
            num_scalar_prefetch=0,
            grid=(B,),
            in_specs=[
                pl.BlockSpec((1, n_pad, K), lambda i: (i, 0, 0)),
                pl.BlockSpec((K, D), lambda i: (0, 0)),
                pl.BlockSpec((1, D), lambda i: (0, 0)),
                pl.BlockSpec((n_pad, D), lambda i: (0, 0)),
                pl.BlockSpec((1, D), lambda i: (0, 0)),
            ],
            out_specs=pl.BlockSpec((1, n_pad, D), lambda i: (i, 0, 0)),
        ),
        compiler_params=pltpu.CompilerParams(
            dimension_semantics=("parallel",),
            vmem_limit_bytes=_vmem_limit(4 * (K * D + 3 * n_pad * D))),
    )(P, patch_embed_w.astype(jnp.bfloat16), _row2d(patch_embed_b), pos_pad, cls0)

    bspec = pl.BlockSpec((1, n_pad, D), lambda i: (i, 0, 0))
    rspec = pl.BlockSpec((1, D), lambda i: (0, 0))
    layer_vmem = _vmem_limit(
        2 * 2 * (3 * D * D + D * D + 2 * D * hidden)
        + 4 * (4 * n_pad * D + n_pad * 3 * D + n_pad * hidden + 2 * n_pad * n_pad))
    for (ln1_g, ln1_b, qkv_w, qkv_b, proj_w, proj_b,
         ln2_g, ln2_b, fc1_w, fc1_b, fc2_w, fc2_b) in blocks:
        xs = pl.pallas_call(
            partial(_layer_kernel, n_tok=n_tok, heads=H, hd=hd, scale=scale),
            out_shape=jax.ShapeDtypeStruct((B, n_pad, D), jnp.float32),
            grid_spec=pltpu.PrefetchScalarGridSpec(
                num_scalar_prefetch=0,
                grid=(B,),
                in_specs=[
                    bspec,
                    rspec,
                    rspec,
                    pl.BlockSpec((D, 3 * D), lambda i: (0, 0)),
                    pl.BlockSpec((1, 3 * D), lambda i: (0, 0)),
                    pl.BlockSpec((D, D), lambda i: (0, 0)),
                    rspec,
                    rspec,
                    rspec,
                    pl.BlockSpec((D, hidden), lambda i: (0, 0)),
                    pl.BlockSpec((1, hidden), lambda i: (0, 0)),
                    pl.BlockSpec((hidden, D), lambda i: (0, 0)),
                    rspec,
                ],
                out_specs=bspec,
            ),
            compiler_params=pltpu.CompilerParams(
                dimension_semantics=("parallel",),
                vmem_limit_bytes=layer_vmem),
        )(xs, _row2d(ln1_g), _row2d(ln1_b),
          qkv_w.astype(jnp.bfloat16), _row2d(qkv_b),
          proj_w.astype(jnp.bfloat16), _row2d(proj_b),
          _row2d(ln2_g), _row2d(ln2_b), fc1_w.astype(jnp.bfloat16),
          _row2d(fc1_b), fc2_w.astype(jnp.bfloat16), _row2d(fc2_b))

    cls_rows = xs[:, 0, :]
    out = pl.pallas_call(
        _final_kernel,
        out_shape=jax.ShapeDtypeStruct((B, D), jnp.float32),
        grid_spec=pltpu.PrefetchScalarGridSpec(
            num_scalar_prefetch=0,
            grid=(1,),
            in_specs=[
                pl.BlockSpec((B, D), lambda i: (0, 0)),
                pl.BlockSpec((1, D), lambda i: (0, 0)),
                pl.BlockSpec((1, D), lambda i: (0, 0)),
            ],
            out_specs=pl.BlockSpec((B, D), lambda i: (0, 0)),
        ),
        compiler_params=pltpu.CompilerParams(
            dimension_semantics=("arbitrary",)),
    )(cls_rows, _row2d(norm_g), _row2d(norm_b))
    return out


# speedup vs baseline: 1.1651x; 1.1651x over previous
import math
from functools import partial

import jax
import jax.numpy as jnp
from jax import lax
from jax.experimental import pallas as pl
from jax.experimental.pallas import tpu as pltpu

_INV_SQRT2 = 1.0 / math.sqrt(2.0)
_NEG_INF = -1e30
_HEADS = 12
_PATCH = 16
_EPS = 1e-5


def _ru(x, m):
    return ((x + m - 1) // m) * m


def _vmem_limit(bytes_needed):
    return int(min(64 * 2**20, max(32 * 2**20, 2 * bytes_needed)))


def _ln_rows(xv, g, b):
    mean = jnp.mean(xv, axis=-1, keepdims=True)
    xc = xv - mean
    var = jnp.mean(xc * xc, axis=-1, keepdims=True)
    return xc * lax.rsqrt(var + _EPS) * g + b


def _embed_kernel(p_ref, w_ref, b_ref, pos_ref, cls_ref, o_ref, *, n_tok):
    t = jnp.dot(p_ref[0], w_ref[...], preferred_element_type=jnp.float32)
    y = t + b_ref[...] + pos_ref[...]
    rows = lax.broadcasted_iota(jnp.int32, (y.shape[0], 1), 0)
    y = jnp.where(rows == 0, cls_ref[...], y)
    y = jnp.where(rows >= n_tok, 0.0, y)
    o_ref[0] = y


def _mha(xv, qkv, mask, *, heads, hd, scale):
    dim = heads * hd
    outs = []
    for h in range(heads):
        q = qkv[:, h * hd:(h + 1) * hd].astype(jnp.bfloat16)
        k = qkv[:, dim + h * hd:dim + (h + 1) * hd].astype(jnp.bfloat16)
        v = qkv[:, 2 * dim + h * hd:2 * dim + (h + 1) * hd].astype(jnp.bfloat16)
        s = lax.dot_general(q, k, (((1,), (1,)), ((), ())),
                            preferred_element_type=jnp.float32) * scale
        s = jnp.where(mask, _NEG_INF, s)
        s = s - jnp.max(s, axis=-1, keepdims=True)
        p = jnp.exp(s)
        p = p / jnp.sum(p, axis=-1, keepdims=True)
        outs.append(jnp.dot(p.astype(jnp.bfloat16), v,
                            preferred_element_type=jnp.float32))
    return jnp.concatenate(outs, axis=1)


def _attn_kernel(x_ref, g_ref, b_ref, wq_ref, bq_ref, o_ref,
                 *, n_tok, heads, hd, scale):
    xv = x_ref[0]
    ln = _ln_rows(xv, g_ref[...], b_ref[...]).astype(jnp.bfloat16)
    qkv = jnp.dot(ln, wq_ref[...], preferred_element_type=jnp.float32)
    qkv = qkv + bq_ref[...]
    mask = lax.broadcasted_iota(jnp.int32, (xv.shape[0], xv.shape[0]), 1) >= n_tok
    o_ref[0] = _mha(xv, qkv, mask, heads=heads, hd=hd, scale=scale)


def _block_kernel(o_ref, x_ref, pw_ref, pb_ref, g_ref, b_ref,
                  w1_ref, b1_ref, w2_ref, b2_ref, out_ref):
    t = jnp.dot(o_ref[0].astype(jnp.bfloat16), pw_ref[...],
                preferred_element_type=jnp.float32) + pb_ref[...]
    xmid = x_ref[0] + t
    ln = _ln_rows(xmid, g_ref[...], b_ref[...]).astype(jnp.bfloat16)
    hh = jnp.dot(ln, w1_ref[...], preferred_element_type=jnp.float32) + b1_ref[...]
    gl = 0.5 * hh * (1.0 + lax.erf(hh * _INV_SQRT2))
    m = jnp.dot(gl.astype(jnp.bfloat16), w2_ref[...],
                preferred_element_type=jnp.float32) + b2_ref[...]
    out_ref[0] = xmid + m


def _final_kernel(x_ref, g_ref, b_ref, o_ref):
    o_ref[...] = _ln_rows(x_ref[...], g_ref[...], b_ref[...])


def _row2d(a):
    return a.reshape(1, a.shape[-1]).astype(jnp.float32)


def kernel(patch_embed_w, patch_embed_b, cls_token, pos_embed, norm_g, norm_b, block0_ln1_g, block0_ln1_b, block0_qkv_w, block0_qkv_b, block0_proj_w, block0_proj_b, block0_ln2_g, block0_ln2_b, block0_fc1_w, block0_fc1_b, block0_fc2_w, block0_fc2_b, block1_ln1_g, block1_ln1_b, block1_qkv_w, block1_qkv_b, block1_proj_w, block1_proj_b, block1_ln2_g, block1_ln2_b, block1_fc1_w, block1_fc1_b, block1_fc2_w, block1_fc2_b, block2_ln1_g, block2_ln1_b, block2_qkv_w, block2_qkv_b, block2_proj_w, block2_proj_b, block2_ln2_g, block2_ln2_b, block2_fc1_w, block2_fc1_b, block2_fc2_w, block2_fc2_b, block3_ln1_g, block3_ln1_b, block3_qkv_w, block3_qkv_b, block3_proj_w, block3_proj_b, block3_ln2_g, block3_ln2_b, block3_fc1_w, block3_fc1_b, block3_fc2_w, block3_fc2_b, block4_ln1_g, block4_ln1_b, block4_qkv_w, block4_qkv_b, block4_proj_w, block4_proj_b, block4_ln2_g, block4_ln2_b, block4_fc1_w, block4_fc1_b, block4_fc2_w, block4_fc2_b, block5_ln1_g, block5_ln1_b, block5_qkv_w, block5_qkv_b, block5_proj_w, block5_proj_b, block5_ln2_g, block5_ln2_b, block5_fc1_w, block5_fc1_b, block5_fc2_w, block5_fc2_b, x):
    blocks = [
        (block0_ln1_g, block0_ln1_b, block0_qkv_w, block0_qkv_b, block0_proj_w,
         block0_proj_b, block0_ln2_g, block0_ln2_b, block0_fc1_w, block0_fc1_b,
         block0_fc2_w, block0_fc2_b),
        (block1_ln1_g, block1_ln1_b, block1_qkv_w, block1_qkv_b, block1_proj_w,
         block1_proj_b, block1_ln2_g, block1_ln2_b, block1_fc1_w, block1_fc1_b,
         block1_fc2_w, block1_fc2_b),
        (block2_ln1_g, block2_ln1_b, block2_qkv_w, block2_qkv_b, block2_proj_w,
         block2_proj_b, block2_ln2_g, block2_ln2_b, block2_fc1_w, block2_fc1_b,
         block2_fc2_w, block2_fc2_b),
        (block3_ln1_g, block3_ln1_b, block3_qkv_w, block3_qkv_b, block3_proj_w,
         block3_proj_b, block3_ln2_g, block3_ln2_b, block3_fc1_w, block3_fc1_b,
         block3_fc2_w, block3_fc2_b),
        (block4_ln1_g, block4_ln1_b, block4_qkv_w, block4_qkv_b, block4_proj_w,
         block4_proj_b, block4_ln2_g, block4_ln2_b, block4_fc1_w, block4_fc1_b,
         block4_fc2_w, block4_fc2_b),
        (block5_ln1_g, block5_ln1_b, block5_qkv_w, block5_qkv_b, block5_proj_w,
         block5_proj_b, block5_ln2_g, block5_ln2_b, block5_fc1_w, block5_fc1_b,
         block5_fc2_w, block5_fc2_b),
    ]

    B, C, IMG, _ = x.shape
    p = _PATCH
    gh = IMG // p
    n_patch = gh * gh
    n_tok = n_patch + 1
    n_pad = _ru(n_tok, 8)
    D = patch_embed_w.shape[1]
    K = C * p * p
    H = _HEADS
    hd = D // H
    hidden = blocks[0][8].shape[1]
    scale = hd ** -0.5

    patches = x.reshape(B, C, gh, p, gh, p).transpose(0, 2, 4, 1, 3, 5)
    patches = patches.reshape(B, n_patch, K)
    P = jnp.pad(patches, ((0, 0), (1, n_pad - n_tok), (0, 0))).astype(jnp.bfloat16)
    pos = pos_embed[0].astype(jnp.float32)
    pos_pad = jnp.pad(pos, ((0, n_pad - n_tok), (0, 0)))
    cls0 = (cls_token[0, 0] + pos[0]).reshape(1, D).astype(jnp.float32)

    xs = pl.pallas_call(
        partial(_embed_kernel, n_tok=n_tok),
        out_shape=jax.ShapeDtypeStruct((B, n_pad, D), jnp.float32),
        grid_spec=pltpu.PrefetchScalarGridSpec(
            num_scalar_prefetch=0,
            grid=(B,),
            in_specs=[
                pl.BlockSpec((1, n_pad, K), lambda i: (i, 0, 0)),
                pl.BlockSpec((K, D), lambda i: (0, 0)),
                pl.BlockSpec((1, D), lambda i: (0, 0)),
                pl.BlockSpec((n_pad, D), lambda i: (0, 0)),
                pl.BlockSpec((1, D), lambda i: (0, 0)),
            ],
            out_specs=pl.BlockSpec((1, n_pad, D), lambda i: (i, 0, 0)),
        ),
        compiler_params=pltpu.CompilerParams(
            dimension_semantics=("parallel",),
            vmem_limit_bytes=_vmem_limit(4 * (K * D + 3 * n_pad * D))),
    )(P, patch_embed_w.astype(jnp.bfloat16), _row2d(patch_embed_b), pos_pad, cls0)

    bspec = pl.BlockSpec((1, n_pad, D), lambda i: (i, 0, 0))
    rspec = pl.BlockSpec((1, D), lambda i: (0, 0))
    for (ln1_g, ln1_b, qkv_w, qkv_b, proj_w, proj_b,
         ln2_g, ln2_b, fc1_w, fc1_b, fc2_w, fc2_b) in blocks:
        o_t = pl.pallas_call(
            partial(_attn_kernel, n_tok=n_tok, heads=H, hd=hd, scale=scale),
            out_shape=jax.ShapeDtypeStruct((B, n_pad, D), jnp.float32),
            grid_spec=pltpu.PrefetchScalarGridSpec(
                num_scalar_prefetch=0,
                grid=(B,),
                in_specs=[
                    bspec,
                    rspec,
                    rspec,
                    pl.BlockSpec((D, 3 * D), lambda i: (0, 0)),
                    pl.BlockSpec((1, 3 * D), lambda i: (0, 0)),
                ],
                out_specs=bspec,
            ),
            compiler_params=pltpu.CompilerParams(
                dimension_semantics=("parallel",),
                vmem_limit_bytes=_vmem_limit(
                    2 * D * 3 * D + 4 * (3 * n_pad * D + n_pad * 3 * D
                                         + 2 * n_pad * n_pad))),
        )(xs, _row2d(ln1_g), _row2d(ln1_b),
          qkv_w.astype(jnp.bfloat16), _row2d(qkv_b))
        xs = pl.pallas_call(
            _block_kernel,
            out_shape=jax.ShapeDtypeStruct((B, n_pad, D), jnp.float32),
            grid_spec=pltpu.PrefetchScalarGridSpec(
                num_scalar_prefetch=0,
                grid=(B,),
                in_specs=[
                    bspec,
                    bspec,
                    pl.BlockSpec((D, D), lambda i: (0, 0)),
                    rspec,
                    rspec,
                    rspec,
                    pl.BlockSpec((D, hidden), lambda i: (0, 0)),
                    pl.BlockSpec((1, hidden), lambda i: (0, 0)),
                    pl.BlockSpec((hidden, D), lambda i: (0, 0)),
                    rspec,
                ],
                out_specs=bspec,
            ),
            compiler_params=pltpu.CompilerParams(
                dimension_semantics=("parallel",),
                vmem_limit_bytes=_vmem_limit(
                    2 * (D * D + 2 * D * hidden)
                    + 4 * (3 * n_pad * D + 3 * n_pad * hidden))),
        )(o_t, xs, proj_w.astype(jnp.bfloat16), _row2d(proj_b),
          _row2d(ln2_g), _row2d(ln2_b), fc1_w.astype(jnp.bfloat16),
          _row2d(fc1_b), fc2_w.astype(jnp.bfloat16), _row2d(fc2_b))

    cls_rows = xs[:, 0, :]
    out = pl.pallas_call(
        _final_kernel,
        out_shape=jax.ShapeDtypeStruct((B, D), jnp.float32),
        grid_spec=pltpu.PrefetchScalarGridSpec(
            num_scalar_prefetch=0,
            grid=(1,),
            in_specs=[
                pl.BlockSpec((B, D), lambda i: (0, 0)),
                pl.BlockSpec((1, D), lambda i: (0, 0)),
                pl.BlockSpec((1, D), lambda i: (0, 0)),
            ],
            out_specs=pl.BlockSpec((B, D), lambda i: (0, 0)),
        ),
        compiler_params=pltpu.CompilerParams(
            dimension_semantics=("arbitrary",)),
    )(cls_rows, _row2d(norm_g), _row2d(norm_b))
    return out


# speedup vs baseline: 1.1892x; 1.0206x over previous
import math
from functools import partial

import jax
import jax.numpy as jnp
from jax import lax
from jax.experimental import pallas as pl
from jax.experimental.pallas import tpu as pltpu

_INV_SQRT2 = 1.0 / math.sqrt(2.0)
_NEG_INF = -1e30
_HEADS = 12
_PATCH = 16
_EPS = 1e-5


def _ru(x, m):
    return ((x + m - 1) // m) * m


def _vmem_limit(bytes_needed):
    return int(min(64 * 2**20, max(32 * 2**20, 2 * bytes_needed)))


def _ln_rows(xv, g, b):
    mean = jnp.mean(xv, axis=-1, keepdims=True)
    xc = xv - mean
    var = jnp.mean(xc * xc, axis=-1, keepdims=True)
    return xc * lax.rsqrt(var + _EPS) * g + b


def _embed_kernel(p_ref, w_ref, b_ref, pos_ref, cls_ref, o_ref, *, n_tok):
    t = jnp.dot(p_ref[0], w_ref[...], preferred_element_type=jnp.float32)
    y = t + b_ref[...] + pos_ref[...]
    rows = lax.broadcasted_iota(jnp.int32, (y.shape[0], 1), 0)
    y = jnp.where(rows == 0, cls_ref[...], y)
    y = jnp.where(rows >= n_tok, 0.0, y)
    o_ref[0] = y


def _mha(qkv, mask, *, nb, n_pad, heads, hd, scale):
    dim = heads * hd
    rows_out = []
    for bi in range(nb):
        r0 = bi * n_pad
        heads_out = []
        for h in range(heads):
            q = qkv[r0:r0 + n_pad, h * hd:(h + 1) * hd].astype(jnp.bfloat16)
            k = qkv[r0:r0 + n_pad,
                    dim + h * hd:dim + (h + 1) * hd].astype(jnp.bfloat16)
            v = qkv[r0:r0 + n_pad,
                    2 * dim + h * hd:2 * dim + (h + 1) * hd].astype(jnp.bfloat16)
            s = lax.dot_general(q, k, (((1,), (1,)), ((), ())),
                                preferred_element_type=jnp.float32) * scale
            s = jnp.where(mask, _NEG_INF, s)
            s = s - jnp.max(s, axis=-1, keepdims=True)
            p = jnp.exp(s)
            p = p / jnp.sum(p, axis=-1, keepdims=True)
            heads_out.append(jnp.dot(p.astype(jnp.bfloat16), v,
                                     preferred_element_type=jnp.float32))
        rows_out.append(jnp.concatenate(heads_out, axis=1))
    return jnp.concatenate(rows_out, axis=0)


_O_LN1G, _O_LN1B, _O_QKVB, _O_PROJB = 0, 768, 1536, 3840
_O_LN2G, _O_LN2B, _O_FC1B, _O_FC2B, _O_END = 4608, 5376, 6144, 9216, 9984


def _attn_kernel(x_ref, p_ref, wq_ref, o_ref,
                 *, nb, n_pad, n_tok, heads, hd, scale):
    xv = x_ref[...].reshape(nb * n_pad, x_ref.shape[-1])
    pv = p_ref[0]
    ln = _ln_rows(xv, pv[:, _O_LN1G:_O_LN1B],
                  pv[:, _O_LN1B:_O_QKVB]).astype(jnp.bfloat16)
    qkv = jnp.dot(ln, wq_ref[...], preferred_element_type=jnp.float32)
    qkv = qkv + pv[:, _O_QKVB:_O_PROJB]
    mask = lax.broadcasted_iota(jnp.int32, (n_pad, n_pad), 1) >= n_tok
    o = _mha(qkv, mask, nb=nb, n_pad=n_pad, heads=heads, hd=hd, scale=scale)
    o_ref[...] = o.astype(jnp.bfloat16).reshape(o_ref.shape)


def _block_kernel(o_ref, x_ref, p_ref, pw_ref, w1_ref, w2_ref, out_ref):
    rows = o_ref.shape[0] * o_ref.shape[1]
    ov = o_ref[...].reshape(rows, o_ref.shape[-1])
    xv = x_ref[...].reshape(rows, x_ref.shape[-1])
    pv = p_ref[0]
    t = jnp.dot(ov, pw_ref[...],
                preferred_element_type=jnp.float32) + pv[:, _O_PROJB:_O_LN2G]
    xmid = xv + t
    ln = _ln_rows(xmid, pv[:, _O_LN2G:_O_LN2B],
                  pv[:, _O_LN2B:_O_FC1B]).astype(jnp.bfloat16)
    hh = jnp.dot(ln, w1_ref[...],
                 preferred_element_type=jnp.float32) + pv[:, _O_FC1B:_O_FC2B]
    gl = 0.5 * hh * (1.0 + lax.erf(hh * _INV_SQRT2))
    m = jnp.dot(gl.astype(jnp.bfloat16), w2_ref[...],
                preferred_element_type=jnp.float32) + pv[:, _O_FC2B:_O_END]
    out_ref[...] = (xmid + m).reshape(out_ref.shape)


def _final_kernel(x_ref, g_ref, b_ref, o_ref):
    o_ref[...] = _ln_rows(x_ref[...], g_ref[...], b_ref[...])


def _row2d(a):
    return a.reshape(1, a.shape[-1]).astype(jnp.float32)


def kernel(patch_embed_w, patch_embed_b, cls_token, pos_embed, norm_g, norm_b, block0_ln1_g, block0_ln1_b, block0_qkv_w, block0_qkv_b, block0_proj_w, block0_proj_b, block0_ln2_g, block0_ln2_b, block0_fc1_w, block0_fc1_b, block0_fc2_w, block0_fc2_b, block1_ln1_g, block1_ln1_b, block1_qkv_w, block1_qkv_b, block1_proj_w, block1_proj_b, block1_ln2_g, block1_ln2_b, block1_fc1_w, block1_fc1_b, block1_fc2_w, block1_fc2_b, block2_ln1_g, block2_ln1_b, block2_qkv_w, block2_qkv_b, block2_proj_w, block2_proj_b, block2_ln2_g, block2_ln2_b, block2_fc1_w, block2_fc1_b, block2_fc2_w, block2_fc2_b, block3_ln1_g, block3_ln1_b, block3_qkv_w, block3_qkv_b, block3_proj_w, block3_proj_b, block3_ln2_g, block3_ln2_b, block3_fc1_w, block3_fc1_b, block3_fc2_w, block3_fc2_b, block4_ln1_g, block4_ln1_b, block4_qkv_w, block4_qkv_b, block4_proj_w, block4_proj_b, block4_ln2_g, block4_ln2_b, block4_fc1_w, block4_fc1_b, block4_fc2_w, block4_fc2_b, block5_ln1_g, block5_ln1_b, block5_qkv_w, block5_qkv_b, block5_proj_w, block5_proj_b, block5_ln2_g, block5_ln2_b, block5_fc1_w, block5_fc1_b, block5_fc2_w, block5_fc2_b, x):
    blocks = [
        (block0_ln1_g, block0_ln1_b, block0_qkv_w, block0_qkv_b, block0_proj_w,
         block0_proj_b, block0_ln2_g, block0_ln2_b, block0_fc1_w, block0_fc1_b,
         block0_fc2_w, block0_fc2_b),
        (block1_ln1_g, block1_ln1_b, block1_qkv_w, block1_qkv_b, block1_proj_w,
         block1_proj_b, block1_ln2_g, block1_ln2_b, block1_fc1_w, block1_fc1_b,
         block1_fc2_w, block1_fc2_b),
        (block2_ln1_g, block2_ln1_b, block2_qkv_w, block2_qkv_b, block2_proj_w,
         block2_proj_b, block2_ln2_g, block2_ln2_b, block2_fc1_w, block2_fc1_b,
         block2_fc2_w, block2_fc2_b),
        (block3_ln1_g, block3_ln1_b, block3_qkv_w, block3_qkv_b, block3_proj_w,
         block3_proj_b, block3_ln2_g, block3_ln2_b, block3_fc1_w, block3_fc1_b,
         block3_fc2_w, block3_fc2_b),
        (block4_ln1_g, block4_ln1_b, block4_qkv_w, block4_qkv_b, block4_proj_w,
         block4_proj_b, block4_ln2_g, block4_ln2_b, block4_fc1_w, block4_fc1_b,
         block4_fc2_w, block4_fc2_b),
        (block5_ln1_g, block5_ln1_b, block5_qkv_w, block5_qkv_b, block5_proj_w,
         block5_proj_b, block5_ln2_g, block5_ln2_b, block5_fc1_w, block5_fc1_b,
         block5_fc2_w, block5_fc2_b),
    ]

    B, C, IMG, _ = x.shape
    p = _PATCH
    gh = IMG // p
    n_patch = gh * gh
    n_tok = n_patch + 1
    n_pad = _ru(n_tok, 8)
    D = patch_embed_w.shape[1]
    K = C * p * p
    H = _HEADS
    hd = D // H
    hidden = blocks[0][8].shape[1]
    scale = hd ** -0.5

    patches = x.reshape(B, C, gh, p, gh, p).transpose(0, 2, 4, 1, 3, 5)
    patches = patches.reshape(B, n_patch, K)
    P = jnp.pad(patches, ((0, 0), (1, n_pad - n_tok), (0, 0))).astype(jnp.bfloat16)
    pos = pos_embed[0].astype(jnp.float32)
    pos_pad = jnp.pad(pos, ((0, n_pad - n_tok), (0, 0)))
    cls0 = (cls_token[0, 0] + pos[0]).reshape(1, D).astype(jnp.float32)

    xs = pl.pallas_call(
        partial(_embed_kernel, n_tok=n_tok),
        out_shape=jax.ShapeDtypeStruct((B, n_pad, D), jnp.float32),
        grid_spec=pltpu.PrefetchScalarGridSpec(
            num_scalar_prefetch=0,
            grid=(B,),
            in_specs=[
                pl.BlockSpec((1, n_pad, K), lambda i: (i, 0, 0)),
                pl.BlockSpec((K, D), lambda i: (0, 0)),
                pl.BlockSpec((1, D), lambda i: (0, 0)),
                pl.BlockSpec((n_pad, D), lambda i: (0, 0)),
                pl.BlockSpec((1, D), lambda i: (0, 0)),
            ],
            out_specs=pl.BlockSpec((1, n_pad, D), lambda i: (i, 0, 0)),
        ),
        compiler_params=pltpu.CompilerParams(
            dimension_semantics=("parallel",),
            vmem_limit_bytes=_vmem_limit(4 * (K * D + 3 * n_pad * D))),
    )(P, patch_embed_w.astype(jnp.bfloat16), _row2d(patch_embed_b), pos_pad, cls0)

    pvec = jnp.concatenate(
        [jnp.concatenate([blk[0], blk[1], blk[3], blk[5], blk[6], blk[7],
                          blk[9], blk[11]]) for blk in blocks]
    ).reshape(len(blocks), 1, _O_END).astype(jnp.float32)

    NB = 2
    bspec = pl.BlockSpec((NB, n_pad, D), lambda i: (i, 0, 0))
    for li, (ln1_g, ln1_b, qkv_w, qkv_b, proj_w, proj_b,
             ln2_g, ln2_b, fc1_w, fc1_b, fc2_w, fc2_b) in enumerate(blocks):
        pspec = pl.BlockSpec((1, 1, _O_END), lambda i, li=li: (li, 0, 0))
        o_t = pl.pallas_call(
            partial(_attn_kernel, nb=NB, n_pad=n_pad, n_tok=n_tok,
                    heads=H, hd=hd, scale=scale),
            out_shape=jax.ShapeDtypeStruct((B, n_pad, D), jnp.bfloat16),
            grid_spec=pltpu.PrefetchScalarGridSpec(
                num_scalar_prefetch=0,
                grid=(B // NB,),
                in_specs=[
                    bspec,
                    pspec,
                    pl.BlockSpec((D, 3 * D), lambda i: (0, 0)),
                ],
                out_specs=bspec,
            ),
            compiler_params=pltpu.CompilerParams(
                dimension_semantics=("parallel",),
                vmem_limit_bytes=_vmem_limit(
                    2 * D * 3 * D + 4 * NB * (3 * n_pad * D + n_pad * 3 * D
                                              + 2 * n_pad * n_pad))),
        )(xs, pvec, qkv_w.astype(jnp.bfloat16))
        xs = pl.pallas_call(
            _block_kernel,
            out_shape=jax.ShapeDtypeStruct((B, n_pad, D), jnp.float32),
            grid_spec=pltpu.PrefetchScalarGridSpec(
                num_scalar_prefetch=0,
                grid=(B // NB,),
                in_specs=[
                    bspec,
                    bspec,
                    pspec,
                    pl.BlockSpec((D, D), lambda i: (0, 0)),
                    pl.BlockSpec((D, hidden), lambda i: (0, 0)),
                    pl.BlockSpec((hidden, D), lambda i: (0, 0)),
                ],
                out_specs=bspec,
            ),
            compiler_params=pltpu.CompilerParams(
                dimension_semantics=("parallel",),
                vmem_limit_bytes=_vmem_limit(
                    2 * (D * D + 2 * D * hidden)
                    + 4 * NB * (3 * n_pad * D + 3 * n_pad * hidden))),
        )(o_t, xs, pvec, proj_w.astype(jnp.bfloat16),
          fc1_w.astype(jnp.bfloat16), fc2_w.astype(jnp.bfloat16))

    cls_rows = xs[:, 0, :]
    out = pl.pallas_call(
        _final_kernel,
        out_shape=jax.ShapeDtypeStruct((B, D), jnp.float32),
        grid_spec=pltpu.PrefetchScalarGridSpec(
            num_scalar_prefetch=0,
            grid=(1,),
            in_specs=[
                pl.BlockSpec((B, D), lambda i: (0, 0)),
                pl.BlockSpec((1, D), lambda i: (0, 0)),
                pl.BlockSpec((1, D), lambda i: (0, 0)),
            ],
            out_specs=pl.BlockSpec((B, D), lambda i: (0, 0)),
        ),
        compiler_params=pltpu.CompilerParams(
            dimension_semantics=("arbitrary",)),
    )(cls_rows, _row2d(norm_g), _row2d(norm_b))
    return out
